# baseline (speedup 1.0000x reference)
import jax
import jax.numpy as jnp
from jax import lax
from jax.experimental import pallas as pl
from jax.experimental.pallas import tpu as pltpu
from jax.experimental.pallas import tpu_sc as plsc

IN_FEATURES = 4096
FEATURES = 128

NC = 2
NS = 16
LANES = 16
NW = NC * NS

CHUNK = 2048
ROWS_PER_CHUNK = CHUNK // FEATURES
GROUPS = FEATURES // LANES
NBUF = 3
UNROLL_ROWS = 1


def _sc_body(x_hbm, rows_hbm, cols_hbm, w_hbm, out_hbm,
             x_v, i0_v, i1_v, i2_v, w0_v, w1_v, w2_v, acc_v,
             sem0, sem1, sem2):
    cid = lax.axis_index("c")
    sid = lax.axis_index("s")
    wid = sid * NC + cid

    pltpu.sync_copy(x_hbm, x_v)

    nchunks_total = rows_hbm.shape[0] // CHUNK
    per = (-(-nchunks_total // NW) + NBUF - 1) // NBUF * NBUF
    start_chunk = wid * per
    n = jnp.clip(nchunks_total - start_chunk, 0, per)

    sems = (sem0, sem1, sem2)
    ibufs = (i0_v, i1_v, i2_v)
    wbufs = (w0_v, w1_v, w2_v)

    def start_fetch(b, c):
        base = (start_chunk + c) * CHUNK
        pltpu.async_copy(rows_hbm.at[pl.ds(base, CHUNK)],
                         ibufs[b].at[pl.ds(0, CHUNK)], sems[b])
        pltpu.async_copy(cols_hbm.at[pl.ds(base, CHUNK)],
                         ibufs[b].at[pl.ds(CHUNK, CHUNK)], sems[b])
        pltpu.async_copy(w_hbm.at[pl.ds(base, CHUNK)], wbufs[b], sems[b])

    def wait_fetch(b):
        pltpu.make_async_copy(rows_hbm.at[pl.ds(0, CHUNK)],
                              ibufs[b].at[pl.ds(0, CHUNK)], sems[b]).wait()
        pltpu.make_async_copy(rows_hbm.at[pl.ds(0, CHUNK)],
                              ibufs[b].at[pl.ds(CHUNK, CHUNK)], sems[b]).wait()
        pltpu.make_async_copy(w_hbm.at[pl.ds(0, CHUNK)],
                              wbufs[b], sems[b]).wait()

    def compute(b, acc):
        ibuf, wbuf = ibufs[b], wbufs[b]

        def rows_body(k, acc):
            accl = list(acc)
            for g in range(GROUPS):
                off = k * FEATURES + g * LANES
                ir = ibuf[pl.ds(off, LANES)]
                ic = ibuf[pl.ds(CHUNK + off, LANES)]
                xr = plsc.load_gather(x_v, [ir])
                xc = plsc.load_gather(x_v, [ic])
                w = wbuf[pl.ds(off, LANES)]
                accl[g] = accl[g] + xr * xc * w
            return tuple(accl)

        return plsc.parallel_loop(
            0, ROWS_PER_CHUNK, 1, unroll=UNROLL_ROWS, carry=acc)(rows_body)

    for b in range(NBUF):
        start_fetch(b, b)

    zero = jnp.zeros((LANES,), jnp.float32)
    acc0 = (zero,) * GROUPS

    def ring_body(k, acc):
        for b in range(NBUF):
            c = NBUF * k + b
            wait_fetch(b)
            acc = compute(b, acc)

            @pl.when(c + NBUF < n)
            def _():
                start_fetch(b, c + NBUF)

        return acc

    acc = lax.fori_loop(0, n // NBUF, ring_body, acc0)

    for g in range(GROUPS):
        acc_v[pl.ds(g * LANES, LANES)] = acc[g]
    pltpu.sync_copy(acc_v, out_hbm.at[wid])


def kernel(x, rows, cols, weights):
    wflat = weights.reshape(-1)
    mesh = plsc.VectorSubcoreMesh(core_axis_name="c", subcore_axis_name="s")
    kfn = pl.kernel(
        _sc_body,
        out_type=jax.ShapeDtypeStruct((NW, FEATURES), jnp.float32),
        mesh=mesh,
        compiler_params=pltpu.CompilerParams(needs_layout_passes=False),
        scratch_types=[
            pltpu.VMEM((IN_FEATURES,), jnp.float32),
            pltpu.VMEM((2 * CHUNK,), jnp.int32),
            pltpu.VMEM((2 * CHUNK,), jnp.int32),
            pltpu.VMEM((2 * CHUNK,), jnp.int32),
            pltpu.VMEM((CHUNK,), jnp.float32),
            pltpu.VMEM((CHUNK,), jnp.float32),
            pltpu.VMEM((CHUNK,), jnp.float32),
            pltpu.VMEM((FEATURES,), jnp.float32),
            pltpu.SemaphoreType.DMA,
            pltpu.SemaphoreType.DMA,
            pltpu.SemaphoreType.DMA,
        ],
    )
    partial = kfn(x, rows, cols, wflat)
    return partial.sum(axis=0)


# speedup vs baseline: 1.0023x; 1.0023x over previous
import jax
import jax.numpy as jnp
from jax import lax
from jax.experimental import pallas as pl
from jax.experimental.pallas import tpu as pltpu
from jax.experimental.pallas import tpu_sc as plsc

IN_FEATURES = 4096
FEATURES = 128

NC = 2
NS = 16
LANES = 16
NW = NC * NS

CHUNK = 2048
ROWS_PER_CHUNK = CHUNK // FEATURES
GROUPS = FEATURES // LANES
NBUF = 3
UNROLL_ROWS = 2


def _sc_body(x_hbm, rows_hbm, cols_hbm, w_hbm, out_hbm,
             x_v, i0_v, i1_v, i2_v, w0_v, w1_v, w2_v, acc_v,
             sem0, sem1, sem2):
    cid = lax.axis_index("c")
    sid = lax.axis_index("s")
    wid = sid * NC + cid

    pltpu.sync_copy(x_hbm, x_v)

    nchunks_total = rows_hbm.shape[0] // CHUNK
    per = (-(-nchunks_total // NW) + NBUF - 1) // NBUF * NBUF
    start_chunk = wid * per
    n = jnp.clip(nchunks_total - start_chunk, 0, per)

    sems = (sem0, sem1, sem2)
    ibufs = (i0_v, i1_v, i2_v)
    wbufs = (w0_v, w1_v, w2_v)

    def start_fetch(b, c):
        base = (start_chunk + c) * CHUNK
        pltpu.async_copy(rows_hbm.at[pl.ds(base, CHUNK)],
                         ibufs[b].at[pl.ds(0, CHUNK)], sems[b])
        pltpu.async_copy(cols_hbm.at[pl.ds(base, CHUNK)],
                         ibufs[b].at[pl.ds(CHUNK, CHUNK)], sems[b])
        pltpu.async_copy(w_hbm.at[pl.ds(base, CHUNK)], wbufs[b], sems[b])

    def wait_fetch(b):
        pltpu.make_async_copy(rows_hbm.at[pl.ds(0, CHUNK)],
                              ibufs[b].at[pl.ds(0, CHUNK)], sems[b]).wait()
        pltpu.make_async_copy(rows_hbm.at[pl.ds(0, CHUNK)],
                              ibufs[b].at[pl.ds(CHUNK, CHUNK)], sems[b]).wait()
        pltpu.make_async_copy(w_hbm.at[pl.ds(0, CHUNK)],
                              wbufs[b], sems[b]).wait()

    def compute(b, acc):
        ibuf, wbuf = ibufs[b], wbufs[b]

        def rows_body(k, acc):
            accl = list(acc)
            for g in range(GROUPS):
                off = k * FEATURES + g * LANES
                ir = ibuf[pl.ds(off, LANES)]
                ic = ibuf[pl.ds(CHUNK + off, LANES)]
                xr = plsc.load_gather(x_v, [ir])
                xc = plsc.load_gather(x_v, [ic])
                w = wbuf[pl.ds(off, LANES)]
                accl[g] = accl[g] + xr * xc * w
            return tuple(accl)

        return plsc.parallel_loop(
            0, ROWS_PER_CHUNK, 1, unroll=UNROLL_ROWS, carry=acc)(rows_body)

    for b in range(NBUF):
        start_fetch(b, b)

    zero = jnp.zeros((LANES,), jnp.float32)
    acc0 = (zero,) * GROUPS

    def ring_body(k, acc):
        for b in range(NBUF):
            c = NBUF * k + b
            wait_fetch(b)
            acc = compute(b, acc)

            @pl.when(c + NBUF < n)
            def _():
                start_fetch(b, c + NBUF)

        return acc

    acc = lax.fori_loop(0, n // NBUF, ring_body, acc0)

    for g in range(GROUPS):
        acc_v[pl.ds(g * LANES, LANES)] = acc[g]
    pltpu.sync_copy(acc_v, out_hbm.at[wid])


def kernel(x, rows, cols, weights):
    wflat = weights.reshape(-1)
    mesh = plsc.VectorSubcoreMesh(core_axis_name="c", subcore_axis_name="s")
    kfn = pl.kernel(
        _sc_body,
        out_type=jax.ShapeDtypeStruct((NW, FEATURES), jnp.float32),
        mesh=mesh,
        compiler_params=pltpu.CompilerParams(needs_layout_passes=False),
        scratch_types=[
            pltpu.VMEM((IN_FEATURES,), jnp.float32),
            pltpu.VMEM((2 * CHUNK,), jnp.int32),
            pltpu.VMEM((2 * CHUNK,), jnp.int32),
            pltpu.VMEM((2 * CHUNK,), jnp.int32),
            pltpu.VMEM((CHUNK,), jnp.float32),
            pltpu.VMEM((CHUNK,), jnp.float32),
            pltpu.VMEM((CHUNK,), jnp.float32),
            pltpu.VMEM((FEATURES,), jnp.float32),
            pltpu.SemaphoreType.DMA,
            pltpu.SemaphoreType.DMA,
            pltpu.SemaphoreType.DMA,
        ],
    )
    partial = kfn(x, rows, cols, wflat)
    return partial.sum(axis=0)
